# baseline (speedup 1.0000x reference)
import jax
import jax.numpy as jnp
from jax.experimental import pallas as pl
from jax.experimental.pallas import tpu as pltpu

N_BINS = 100
BW = 0.01
B, H, W = 4, 512, 512
CR = 512
NCHUNK = (B * H) // CR
CHUNKS_PER_BATCH = H // CR
N_KNOTS = N_BINS + 2


def _psi(a, s_ref):
    rows = a.shape[0]
    p1 = a * (1.0 / BW) + 0.5
    j = jax.lax.broadcasted_iota(
        jnp.int32, (1, N_KNOTS, W), 1).astype(jnp.float32)
    ramp = jnp.maximum(p1[:, None, :] - j, 0.0)
    folded = ((ramp[:, :, 0:128] + ramp[:, :, 128:256])
              + (ramp[:, :, 256:384] + ramp[:, :, 384:512]))
    s_ref[0:rows, :] = jnp.sum(folded, axis=-1)


def _pdf(s):
    h = s[:, :N_BINS] - 2.0 * s[:, 1:N_BINS + 1] + s[:, 2:N_BINS + 2]
    return h / (jnp.sum(h, axis=-1, keepdims=True) + 1e-10)


def _loss_kernel(s1_ref, s2_ref, xc_ref, tc_ref, s3_ref, x_hbm, t_hbm,
                 out_ref, bufx, buft, sx_ref, st_ref, semx, semt):
    def start(c, slot):
        pltpu.make_async_copy(
            x_hbm.at[pl.ds(c * CR, CR), :], bufx.at[slot], semx.at[slot]
        ).start()
        pltpu.make_async_copy(
            t_hbm.at[pl.ds(c * CR, CR), :], buft.at[slot], semt.at[slot]
        ).start()

    def chunk_partial(c, slot):
        pltpu.make_async_copy(bufx.at[slot], bufx.at[slot], semx.at[slot]).wait()
        pltpu.make_async_copy(buft.at[slot], buft.at[slot], semt.at[slot]).wait()
        x = bufx[slot]
        t = buft[slot]
        b = c // CHUNKS_PER_BATCH

        diffsum = jnp.sum(jnp.abs(x - t))
        _psi(x, sx_ref)
        _psi(t, st_ref)
        pn = _pdf(sx_ref[...]) + 1e-5
        pc = _pdf(st_ref[...]) + 1e-5
        kl = jnp.sum(pc * (jnp.log(pc) - jnp.log(pn)))
        return s1_ref[b] * diffsum + s2_ref[b] * kl

    start(0, 0)

    def body(i, acc):
        c0 = 2 * i
        start(c0 + 1, 1)
        acc = acc + chunk_partial(c0, 0)

        @pl.when(i + 1 < NCHUNK // 2)
        def _():
            start(c0 + 2, 0)

        return acc + chunk_partial(c0 + 1, 1)

    acc = jax.lax.fori_loop(0, NCHUNK // 2, body, jnp.float32(0.0))

    _psi(xc_ref[...], sx_ref)
    _psi(tc_ref[...], st_ref)
    pnc = _pdf(sx_ref[0:B, :]) + 1e-6
    pcc = _pdf(st_ref[0:B, :]) + 1e-6
    klc = jnp.sum(pcc * (jnp.log(pcc) - jnp.log(pnc)), axis=-1,
                  keepdims=True)
    acc = acc + jnp.sum(s3_ref[...] * klc)
    out_ref[...] = acc.reshape(1, 1)


def kernel(inputo, target, we1, we2, we3):
    eps = 1e-6
    w1 = we1.reshape(B) + eps
    w2 = we2.reshape(B) + eps
    w3 = we3.reshape(B) + eps
    n_total = B * H * W
    s1 = (w1 + 1.0 / w1) / n_total
    s2 = (w2 + 1.0 / w2) / (2 * B * H)
    s3 = ((w3 + 1.0 / w3) / (2 * B * H)).reshape(B, 1)

    x = inputo.reshape(B * H, W)
    t = target.reshape(B * H, W)
    xc = inputo[:, 0, :, 0]
    tc = target[:, 0, :, 0]

    out = pl.pallas_call(
        _loss_kernel,
        out_shape=jax.ShapeDtypeStruct((1, 1), jnp.float32),
        in_specs=[
            pl.BlockSpec(memory_space=pltpu.SMEM),
            pl.BlockSpec(memory_space=pltpu.SMEM),
            pl.BlockSpec((B, H), lambda: (0, 0)),
            pl.BlockSpec((B, H), lambda: (0, 0)),
            pl.BlockSpec((B, 1), lambda: (0, 0)),
            pl.BlockSpec(memory_space=pl.ANY),
            pl.BlockSpec(memory_space=pl.ANY),
        ],
        out_specs=pl.BlockSpec((1, 1), lambda: (0, 0)),
        scratch_shapes=[
            pltpu.VMEM((2, CR, W), jnp.float32),
            pltpu.VMEM((2, CR, W), jnp.float32),
            pltpu.VMEM((CR, N_KNOTS), jnp.float32),
            pltpu.VMEM((CR, N_KNOTS), jnp.float32),
            pltpu.SemaphoreType.DMA((2,)),
            pltpu.SemaphoreType.DMA((2,)),
        ],
        name="customloss_kll",
    )(s1, s2, xc, tc, s3, x, t)

    return out[0, 0]


# speedup vs baseline: 1.0813x; 1.0813x over previous
import jax
import jax.numpy as jnp
from jax.experimental import pallas as pl
from jax.experimental.pallas import tpu as pltpu

N_BINS = 100
BW = 0.01
B, H, W = 4, 512, 512
CR = 256
NCHUNK = (B * H) // CR
CHUNKS_PER_BATCH = H // CR
N_KNOTS = N_BINS + 2


def _psi(a, s_ref):
    rows = a.shape[0]
    p1 = a * (1.0 / BW) + 0.5
    j = jax.lax.broadcasted_iota(
        jnp.int32, (1, N_KNOTS, W), 1).astype(jnp.float32)
    ramp = jnp.maximum(p1[:, None, :] - j, 0.0)
    folded = ((ramp[:, :, 0:128] + ramp[:, :, 128:256])
              + (ramp[:, :, 256:384] + ramp[:, :, 384:512]))
    s_ref[0:rows, :] = jnp.sum(folded, axis=-1)


def _pdf(s):
    h = s[:, :N_BINS] - 2.0 * s[:, 1:N_BINS + 1] + s[:, 2:N_BINS + 2]
    return h / (jnp.sum(h, axis=-1, keepdims=True) + 1e-10)


def _loss_kernel(s1_ref, s2_ref, xc_ref, tc_ref, s3_ref, x_hbm, t_hbm,
                 out_ref, bufx, buft, sx_ref, st_ref, semx, semt):
    def start(c, slot):
        pltpu.make_async_copy(
            x_hbm.at[pl.ds(c * CR, CR), :], bufx.at[slot], semx.at[slot]
        ).start()
        pltpu.make_async_copy(
            t_hbm.at[pl.ds(c * CR, CR), :], buft.at[slot], semt.at[slot]
        ).start()

    def chunk_partial(c, slot):
        pltpu.make_async_copy(bufx.at[slot], bufx.at[slot], semx.at[slot]).wait()
        pltpu.make_async_copy(buft.at[slot], buft.at[slot], semt.at[slot]).wait()
        x = bufx[slot]
        t = buft[slot]
        b = c // CHUNKS_PER_BATCH

        diffsum = jnp.sum(jnp.abs(x - t))
        _psi(x, sx_ref)
        _psi(t, st_ref)
        pn = _pdf(sx_ref[...]) + 1e-5
        pc = _pdf(st_ref[...]) + 1e-5
        kl = jnp.sum(pc * (jnp.log(pc) - jnp.log(pn)))
        return s1_ref[b] * diffsum + s2_ref[b] * kl

    start(0, 0)

    def body(i, acc):
        c0 = 2 * i
        start(c0 + 1, 1)
        acc = acc + chunk_partial(c0, 0)

        @pl.when(i + 1 < NCHUNK // 2)
        def _():
            start(c0 + 2, 0)

        return acc + chunk_partial(c0 + 1, 1)

    acc = jax.lax.fori_loop(0, NCHUNK // 2, body, jnp.float32(0.0))

    _psi(xc_ref[...], sx_ref)
    _psi(tc_ref[...], st_ref)
    pnc = _pdf(sx_ref[0:B, :]) + 1e-6
    pcc = _pdf(st_ref[0:B, :]) + 1e-6
    klc = jnp.sum(pcc * (jnp.log(pcc) - jnp.log(pnc)), axis=-1,
                  keepdims=True)
    acc = acc + jnp.sum(s3_ref[...] * klc)
    out_ref[...] = acc.reshape(1, 1)


def kernel(inputo, target, we1, we2, we3):
    eps = 1e-6
    w1 = we1.reshape(B) + eps
    w2 = we2.reshape(B) + eps
    w3 = we3.reshape(B) + eps
    n_total = B * H * W
    s1 = (w1 + 1.0 / w1) / n_total
    s2 = (w2 + 1.0 / w2) / (2 * B * H)
    s3 = ((w3 + 1.0 / w3) / (2 * B * H)).reshape(B, 1)

    x = inputo.reshape(B * H, W)
    t = target.reshape(B * H, W)
    xc = inputo[:, 0, :, 0]
    tc = target[:, 0, :, 0]

    out = pl.pallas_call(
        _loss_kernel,
        out_shape=jax.ShapeDtypeStruct((1, 1), jnp.float32),
        in_specs=[
            pl.BlockSpec(memory_space=pltpu.SMEM),
            pl.BlockSpec(memory_space=pltpu.SMEM),
            pl.BlockSpec((B, H), lambda: (0, 0)),
            pl.BlockSpec((B, H), lambda: (0, 0)),
            pl.BlockSpec((B, 1), lambda: (0, 0)),
            pl.BlockSpec(memory_space=pl.ANY),
            pl.BlockSpec(memory_space=pl.ANY),
        ],
        out_specs=pl.BlockSpec((1, 1), lambda: (0, 0)),
        scratch_shapes=[
            pltpu.VMEM((2, CR, W), jnp.float32),
            pltpu.VMEM((2, CR, W), jnp.float32),
            pltpu.VMEM((CR, N_KNOTS), jnp.float32),
            pltpu.VMEM((CR, N_KNOTS), jnp.float32),
            pltpu.SemaphoreType.DMA((2,)),
            pltpu.SemaphoreType.DMA((2,)),
        ],
        name="customloss_kll",
    )(s1, s2, xc, tc, s3, x, t)

    return out[0, 0]


# speedup vs baseline: 1.0817x; 1.0003x over previous
import jax
import jax.numpy as jnp
from jax.experimental import pallas as pl
from jax.experimental.pallas import tpu as pltpu

N_BINS = 100
BW = 0.01
B, H, W = 4, 512, 512
CR = 256
NCHUNK = (B * H) // CR
CHUNKS_PER_BATCH = H // CR
N_KNOTS = N_BINS + 2


def _psi(a, s_ref):
    rows = a.shape[0]
    p1 = a * (1.0 / BW) + 0.5
    j = jax.lax.broadcasted_iota(
        jnp.int32, (1, N_KNOTS, W), 1).astype(jnp.float32)
    ramp = jnp.maximum(p1[:, None, :] - j, 0.0)
    folded = ((ramp[:, :, 0:128] + ramp[:, :, 128:256])
              + (ramp[:, :, 256:384] + ramp[:, :, 384:512]))
    s_ref[0:rows, :] = jnp.sum(folded, axis=-1)


def _pdf(s):
    h = s[:, :N_BINS] - 2.0 * s[:, 1:N_BINS + 1] + s[:, 2:N_BINS + 2]
    return h / (jnp.sum(h, axis=-1, keepdims=True) + 1e-10)


def _loss_kernel(s1_ref, s2_ref, xc_ref, tc_ref, s3_ref, x_hbm, t_hbm,
                 out_ref, bufx, buft, sx_ref, st_ref, semx, semt):
    def start(c, slot):
        pltpu.make_async_copy(
            x_hbm.at[pl.ds(c * CR, CR), :], bufx.at[slot], semx.at[slot]
        ).start()
        pltpu.make_async_copy(
            t_hbm.at[pl.ds(c * CR, CR), :], buft.at[slot], semt.at[slot]
        ).start()

    def chunk_partial(c, slot):
        pltpu.make_async_copy(bufx.at[slot], bufx.at[slot], semx.at[slot]).wait()
        pltpu.make_async_copy(buft.at[slot], buft.at[slot], semt.at[slot]).wait()
        x = bufx[slot]
        t = buft[slot]
        b = c // CHUNKS_PER_BATCH

        diffsum = jnp.sum(jnp.abs(x - t))
        _psi(x, sx_ref)
        _psi(t, st_ref)
        pn = _pdf(sx_ref[...]) + 1e-5
        pc = _pdf(st_ref[...]) + 1e-5
        kl = jnp.sum(pc * (jnp.log(pc) - jnp.log(pn)))
        return s1_ref[b] * diffsum + s2_ref[b] * kl

    start(0, 0)

    def body(i, acc):
        c0 = 2 * i
        start(c0 + 1, 1)
        acc = acc + chunk_partial(c0, 0)

        @pl.when(i + 1 < NCHUNK // 2)
        def _():
            start(c0 + 2, 0)

        return acc + chunk_partial(c0 + 1, 1)

    acc = jax.lax.fori_loop(0, NCHUNK // 2, body, jnp.float32(0.0))

    _psi(xc_ref[...], sx_ref)
    _psi(tc_ref[...], st_ref)
    pnc = _pdf(sx_ref[0:B, :]) + 1e-6
    pcc = _pdf(st_ref[0:B, :]) + 1e-6
    klc = jnp.sum(pcc * (jnp.log(pcc) - jnp.log(pnc)), axis=-1,
                  keepdims=True)
    acc = acc + jnp.sum(s3_ref[...] * klc)
    out_ref[...] = acc.reshape(1, 1)


def kernel(inputo, target, we1, we2, we3):
    eps = 1e-6
    w1 = we1.reshape(B) + eps
    w2 = we2.reshape(B) + eps
    w3 = we3.reshape(B) + eps
    n_total = B * H * W
    s1 = (w1 + 1.0 / w1) / n_total
    s2 = (w2 + 1.0 / w2) / (2 * B * H)
    s3 = ((w3 + 1.0 / w3) / (2 * B * H)).reshape(B, 1)

    x = inputo.reshape(B * H, W)
    t = target.reshape(B * H, W)
    xc = inputo[:, 0, :, 0]
    tc = target[:, 0, :, 0]

    out = pl.pallas_call(
        _loss_kernel,
        out_shape=jax.ShapeDtypeStruct((1, 1), jnp.float32),
        in_specs=[
            pl.BlockSpec(memory_space=pltpu.SMEM),
            pl.BlockSpec(memory_space=pltpu.SMEM),
            pl.BlockSpec((B, H), lambda: (0, 0)),
            pl.BlockSpec((B, H), lambda: (0, 0)),
            pl.BlockSpec((B, 1), lambda: (0, 0)),
            pl.BlockSpec(memory_space=pl.ANY),
            pl.BlockSpec(memory_space=pl.ANY),
        ],
        out_specs=pl.BlockSpec((1, 1), lambda: (0, 0)),
        scratch_shapes=[
            pltpu.VMEM((2, CR, W), jnp.float32),
            pltpu.VMEM((2, CR, W), jnp.float32),
            pltpu.VMEM((CR, N_KNOTS), jnp.float32),
            pltpu.VMEM((CR, N_KNOTS), jnp.float32),
            pltpu.SemaphoreType.DMA((2,)),
            pltpu.SemaphoreType.DMA((2,)),
        ],
        name="customloss_kll",
    )(s1, s2, xc, tc, s3, x, t)

    return out.reshape(())


# speedup vs baseline: 1.0884x; 1.0062x over previous
import jax
import jax.numpy as jnp
from jax.experimental import pallas as pl
from jax.experimental.pallas import tpu as pltpu

N_BINS = 100
BW = 0.01
B, H, W = 4, 512, 512
CR = 256
NCHUNK = (B * H) // CR
CHUNKS_PER_BATCH = H // CR
N_KNOTS = N_BINS + 4


def _psi(a, s_ref):
    rows = a.shape[0]
    p1 = a * (1.0 / BW) + 0.5
    j = jax.lax.broadcasted_iota(
        jnp.int32, (1, N_KNOTS, W), 1).astype(jnp.float32)
    ramp = jnp.maximum(p1[:, None, :] - j, 0.0)
    folded = ((ramp[:, :, 0:128] + ramp[:, :, 128:256])
              + (ramp[:, :, 256:384] + ramp[:, :, 384:512]))
    s_ref[0:rows, :] = jnp.sum(folded, axis=-1)


def _pdf(s):
    h = s[:, :N_BINS] - 2.0 * s[:, 1:N_BINS + 1] + s[:, 2:N_BINS + 2]
    return h / (jnp.sum(h, axis=-1, keepdims=True) + 1e-10)


def _loss_kernel(s1_ref, s2_ref, xc_ref, tc_ref, s3_ref, x_hbm, t_hbm,
                 out_ref, bufx, buft, sx_ref, st_ref, semx, semt):
    def start(c, slot):
        pltpu.make_async_copy(
            x_hbm.at[pl.ds(c * CR, CR), :], bufx.at[slot], semx.at[slot]
        ).start()
        pltpu.make_async_copy(
            t_hbm.at[pl.ds(c * CR, CR), :], buft.at[slot], semt.at[slot]
        ).start()

    def chunk_partial(c, slot):
        pltpu.make_async_copy(bufx.at[slot], bufx.at[slot], semx.at[slot]).wait()
        pltpu.make_async_copy(buft.at[slot], buft.at[slot], semt.at[slot]).wait()
        x = bufx[slot]
        t = buft[slot]
        b = c // CHUNKS_PER_BATCH

        diffsum = jnp.sum(jnp.abs(x - t))
        _psi(x, sx_ref)
        _psi(t, st_ref)
        pn = _pdf(sx_ref[...]) + 1e-5
        pc = _pdf(st_ref[...]) + 1e-5
        kl = jnp.sum(pc * (jnp.log(pc) - jnp.log(pn)))
        return s1_ref[b] * diffsum + s2_ref[b] * kl

    start(0, 0)

    def body(i, acc):
        c0 = 2 * i
        start(c0 + 1, 1)
        acc = acc + chunk_partial(c0, 0)

        @pl.when(i + 1 < NCHUNK // 2)
        def _():
            start(c0 + 2, 0)

        return acc + chunk_partial(c0 + 1, 1)

    acc = jax.lax.fori_loop(0, NCHUNK // 2, body, jnp.float32(0.0))

    _psi(xc_ref[...], sx_ref)
    _psi(tc_ref[...], st_ref)
    pnc = _pdf(sx_ref[0:B, :]) + 1e-6
    pcc = _pdf(st_ref[0:B, :]) + 1e-6
    klc = jnp.sum(pcc * (jnp.log(pcc) - jnp.log(pnc)), axis=-1,
                  keepdims=True)
    acc = acc + jnp.sum(s3_ref[...] * klc)
    out_ref[...] = acc.reshape(1, 1)


def kernel(inputo, target, we1, we2, we3):
    eps = 1e-6
    w1 = we1.reshape(B) + eps
    w2 = we2.reshape(B) + eps
    w3 = we3.reshape(B) + eps
    n_total = B * H * W
    s1 = (w1 + 1.0 / w1) / n_total
    s2 = (w2 + 1.0 / w2) / (2 * B * H)
    s3 = ((w3 + 1.0 / w3) / (2 * B * H)).reshape(B, 1)

    x = inputo.reshape(B * H, W)
    t = target.reshape(B * H, W)
    xc = inputo[:, 0, :, 0]
    tc = target[:, 0, :, 0]

    out = pl.pallas_call(
        _loss_kernel,
        out_shape=jax.ShapeDtypeStruct((1, 1), jnp.float32),
        in_specs=[
            pl.BlockSpec(memory_space=pltpu.SMEM),
            pl.BlockSpec(memory_space=pltpu.SMEM),
            pl.BlockSpec((B, H), lambda: (0, 0)),
            pl.BlockSpec((B, H), lambda: (0, 0)),
            pl.BlockSpec((B, 1), lambda: (0, 0)),
            pl.BlockSpec(memory_space=pl.ANY),
            pl.BlockSpec(memory_space=pl.ANY),
        ],
        out_specs=pl.BlockSpec((1, 1), lambda: (0, 0)),
        scratch_shapes=[
            pltpu.VMEM((2, CR, W), jnp.float32),
            pltpu.VMEM((2, CR, W), jnp.float32),
            pltpu.VMEM((CR, N_KNOTS), jnp.float32),
            pltpu.VMEM((CR, N_KNOTS), jnp.float32),
            pltpu.SemaphoreType.DMA((2,)),
            pltpu.SemaphoreType.DMA((2,)),
        ],
        name="customloss_kll",
    )(s1, s2, xc, tc, s3, x, t)

    return out.reshape(())
